# baseline (speedup 1.0000x reference)
import functools

import jax
import jax.numpy as jnp
from jax import lax
from jax.experimental import pallas as pl
from jax.experimental.pallas import tpu as pltpu
from jax.experimental.pallas import tpu_sc as plsc

N = 10000
E = 320000
D = 128
H = 128
C = 10
G = 128
EPS = 1e-5

NC = 2
NS = 16
NW = NC * NS
NPAD = 10240
EW = E // NW
K = 80
STEPS = EW // K


def _scale_rows(rows, sbuf):
    @pl.loop(0, K // 16)
    def _(t):
        s16 = sbuf[pl.ds(t * 16, 16)]
        for l in range(16):
            sc = s16[l]
            for j in range(8):
                rows[t * 16 + l, pl.ds(j * 16, 16)] = (
                    rows[t * 16 + l, pl.ds(j * 16, 16)] * sc)


def _msg_body(scaled, *refs):
    if scaled:
        (h_hbm, row_hbm, col_hbm, ew_hbm, out_hbm, acc_sh,
         ridx_a, cidx_a, sbuf_a, ridx_b, cidx_b, sbuf_b, rows_a, rows_b,
         isem_a, isem_b, gsem_a, gsem_b, ssem_a, ssem_b) = refs
    else:
        (h_hbm, row_hbm, col_hbm, out_hbm, acc_sh,
         ridx_a, cidx_a, ridx_b, cidx_b, rows_a, rows_b,
         isem_a, isem_b, gsem_a, gsem_b, ssem_a, ssem_b) = refs
    cid = lax.axis_index("c")
    sid = lax.axis_index("s")
    wid = sid * NC + cid
    @pl.loop(0, K)
    def _(r):
        for j in range(8):
            rows_a[r, pl.ds(j * 16, 16)] = jnp.zeros((16,), jnp.float32)
    slab = NPAD // NS
    @pl.loop(0, slab // K)
    def _(t):
        pltpu.sync_copy(rows_a, acc_sh.at[pl.ds(sid * slab + t * K, K), :])
    plsc.subcore_barrier()

    def fetch_idx(base, ridx, cidx, sbuf, sem):
        ds = [pltpu.async_copy(row_hbm.at[pl.ds(base, K)], ridx, sem),
              pltpu.async_copy(col_hbm.at[pl.ds(base, K)], cidx, sem)]
        if scaled:
            ds.append(pltpu.async_copy(ew_hbm.at[pl.ds(base, K)], sbuf, sem))
        return ds

    def drain(ds):
        for d in ds:
            d.wait()

    base0 = wid * EW
    @pl.loop(0, STEPS // 2)
    def _(ip):
        base_a = base0 + ip * (2 * K)
        base_b = base_a + K
        ia = fetch_idx(base_a, ridx_a, cidx_a, sbuf_a if scaled else None,
                       isem_a)
        ib = fetch_idx(base_b, ridx_b, cidx_b, sbuf_b if scaled else None,
                       isem_b)
        drain(ia)
        ga = pltpu.async_copy(h_hbm.at[ridx_a], rows_a, gsem_a)
        drain(ib)
        gb = pltpu.async_copy(h_hbm.at[ridx_b], rows_b, gsem_b)
        ga.wait()
        if scaled:
            _scale_rows(rows_a, sbuf_a)
        sa = pltpu.async_copy(rows_a, acc_sh.at[cidx_a], ssem_a, add=True)
        gb.wait()
        if scaled:
            _scale_rows(rows_b, sbuf_b)
        sb = pltpu.async_copy(rows_b, acc_sh.at[cidx_b], ssem_b, add=True)
        sa.wait()
        sb.wait()
    if STEPS % 2:
        base_t = base0 + (STEPS - 1) * K
        drain(fetch_idx(base_t, ridx_a, cidx_a, sbuf_a if scaled else None,
                        isem_a))
        pltpu.async_copy(h_hbm.at[ridx_a], rows_a, gsem_a).wait()
        if scaled:
            _scale_rows(rows_a, sbuf_a)
        pltpu.sync_copy(rows_a, acc_sh.at[cidx_a], add=True)
    plsc.subcore_barrier()
    pltpu.sync_copy(acc_sh.at[pl.ds(sid * slab, slab), :],
                    out_hbm.at[cid, pl.ds(sid * slab, slab), :])


def _attn_body(u_hbm, v_hbm, row_hbm, col_hbm,
               ewc_hbm, ewo_hbm, deg_hbm,
               u_vmem, v_vmem, dc_vmem, do_vmem, ridx, cidx, wc_buf, wo_buf):
    cid = lax.axis_index("c")
    sid = lax.axis_index("s")
    wid = sid * NC + cid
    pltpu.sync_copy(u_hbm, u_vmem)
    pltpu.sync_copy(v_hbm, v_vmem)
    @pl.loop(0, NPAD // 16)
    def _(t):
        dc_vmem[pl.ds(t * 16, 16)] = jnp.zeros((16,), jnp.float32)
        do_vmem[pl.ds(t * 16, 16)] = jnp.zeros((16,), jnp.float32)
    base0 = wid * EW
    @pl.loop(0, STEPS)
    def _(i):
        base = base0 + i * K
        pltpu.sync_copy(row_hbm.at[pl.ds(base, K)], ridx)
        pltpu.sync_copy(col_hbm.at[pl.ds(base, K)], cidx)
        @pl.loop(0, K // 16)
        def _(t):
            r16 = ridx[pl.ds(t * 16, 16)]
            c16 = cidx[pl.ds(t * 16, 16)]
            s = plsc.load_gather(u_vmem, [r16]) + plsc.load_gather(v_vmem, [c16])
            wc = 1.0 / (1.0 + jnp.exp(-s))
            wo = 1.0 - wc
            wc_buf[pl.ds(t * 16, 16)] = wc
            wo_buf[pl.ds(t * 16, 16)] = wo
            plsc.addupdate_scatter(dc_vmem, [r16], wc)
            plsc.addupdate_scatter(do_vmem, [r16], wo)
        pltpu.sync_copy(wc_buf, ewc_hbm.at[pl.ds(base, K)])
        pltpu.sync_copy(wo_buf, ewo_hbm.at[pl.ds(base, K)])
    pltpu.sync_copy(dc_vmem, deg_hbm.at[wid, 0])
    pltpu.sync_copy(do_vmem, deg_hbm.at[wid, 1])


@jax.jit
def _sc_attn(u_pad, v_pad, row, col):
    mesh = plsc.VectorSubcoreMesh(core_axis_name="c", subcore_axis_name="s")
    kern = pl.kernel(
        _attn_body,
        compiler_params=pltpu.CompilerParams(needs_layout_passes=False),
        out_type=(
            jax.ShapeDtypeStruct((E,), jnp.float32),
            jax.ShapeDtypeStruct((E,), jnp.float32),
            jax.ShapeDtypeStruct((NW, 2, NPAD), jnp.float32),
        ),
        mesh=mesh,
        scratch_types=[
            pltpu.VMEM((NPAD,), jnp.float32),
            pltpu.VMEM((NPAD,), jnp.float32),
            pltpu.VMEM((NPAD,), jnp.float32),
            pltpu.VMEM((NPAD,), jnp.float32),
            pltpu.VMEM((K,), jnp.int32),
            pltpu.VMEM((K,), jnp.int32),
            pltpu.VMEM((K,), jnp.float32),
            pltpu.VMEM((K,), jnp.float32),
        ],
    )
    return kern(u_pad, v_pad, row, col)


@jax.jit
def _sc_msgpass(h_pad, row, col):
    mesh = plsc.VectorSubcoreMesh(core_axis_name="c", subcore_axis_name="s")
    kern = pl.kernel(
        functools.partial(_msg_body, False),
        out_type=jax.ShapeDtypeStruct((NC, NPAD, 128), jnp.float32),
        mesh=mesh,
        scratch_types=[
            pltpu.VMEM_SHARED((NPAD, 128), jnp.float32),
            pltpu.VMEM((K,), jnp.int32),
            pltpu.VMEM((K,), jnp.int32),
            pltpu.VMEM((K,), jnp.int32),
            pltpu.VMEM((K,), jnp.int32),
            pltpu.VMEM((K, 128), jnp.float32),
            pltpu.VMEM((K, 128), jnp.float32),
        ] + [pltpu.SemaphoreType.DMA] * 6,
    )
    return kern(h_pad, row, col)


@jax.jit
def _sc_msgpass_scaled(h_pad, row, col, ew):
    mesh = plsc.VectorSubcoreMesh(core_axis_name="c", subcore_axis_name="s")
    kern = pl.kernel(
        functools.partial(_msg_body, True),
        out_type=jax.ShapeDtypeStruct((NC, NPAD, 128), jnp.float32),
        mesh=mesh,
        scratch_types=[
            pltpu.VMEM_SHARED((NPAD, 128), jnp.float32),
            pltpu.VMEM((K,), jnp.int32),
            pltpu.VMEM((K,), jnp.int32),
            pltpu.VMEM((K,), jnp.float32),
            pltpu.VMEM((K,), jnp.int32),
            pltpu.VMEM((K,), jnp.int32),
            pltpu.VMEM((K,), jnp.float32),
            pltpu.VMEM((K, 128), jnp.float32),
            pltpu.VMEM((K, 128), jnp.float32),
        ] + [pltpu.SemaphoreType.DMA] * 6,
    )
    return kern(h_pad, row, col, ew)


def _bn(x, g, b):
    m = jnp.mean(x, axis=0)
    v = jnp.mean(x * x, axis=0) - m * m
    return (x - m) * lax.rsqrt(v + EPS) * g + b


def _log_softmax(z):
    zm = z - jnp.max(z, axis=-1, keepdims=True)
    return zm - jnp.log(jnp.sum(jnp.exp(zm), axis=-1, keepdims=True))


def _head(z, p, pre):
    z = _bn(z, p[pre + "1bn_g"], p[pre + "1bn_b"])
    z = jax.nn.relu(z @ p[pre + "1_W"] + p[pre + "1_b"])
    z = _bn(z, p[pre + "2bn_g"], p[pre + "2bn_b"])
    z = z @ p[pre + "2_W"] + p[pre + "2_b"]
    return _log_softmax(z)


def _pool_heads_body(xc_ref, xo_ref, batch_ref, *rest):
    (hp_refs, outc_ref, outo_ref, outco_ref) = (rest[:-3], rest[-3], rest[-2], rest[-1])
    names = _HEAD_PARAM_NAMES
    p = {k: r[...] for k, r in zip(names, hp_refs)}
    onehot = (batch_ref[0:1, :] == lax.broadcasted_iota(jnp.int32, (G, N), 0))
    onehot = onehot.astype(jnp.float32)
    pc = jnp.dot(onehot, xc_ref[...], preferred_element_type=jnp.float32)
    po = jnp.dot(onehot, xo_ref[...], preferred_element_type=jnp.float32)
    outc_ref[...] = _head(pc, p, "c")
    outo_ref[...] = _head(po, p, "o")
    outco_ref[...] = _head(pc + po, p, "co")


_HEAD_PARAM_NAMES = tuple(
    pre + suf
    for pre in ("c", "o", "co")
    for suf in ("1bn_g", "1bn_b", "1_W", "1_b", "2bn_g", "2bn_b", "2_W", "2_b")
)


def _pool_and_heads(xc, xo, batch, params):
    hp = [params[k] for k in _HEAD_PARAM_NAMES]
    out_shape = [jax.ShapeDtypeStruct((G, C), jnp.float32)] * 3
    outs = pl.pallas_call(
        _pool_heads_body,
        out_shape=out_shape,
    )(xc, xo, batch.reshape(1, N), *hp)
    return outs


def _tc_feat(x, g, b, W, wb):
    def body(x_ref, g_ref, b_ref, W_ref, wb_ref, o_ref):
        h = _bn(x_ref[...], g_ref[...], b_ref[...])
        o_ref[...] = jax.nn.relu(
            jnp.dot(h, W_ref[...], preferred_element_type=jnp.float32)
            + wb_ref[...])
    return pl.pallas_call(
        body, out_shape=jax.ShapeDtypeStruct((N, H), jnp.float32),
    )(x, g, b, W, wb)


def _tc_gin_dense(h, mp, W1, b1, g1, be1, W2, b2):
    def body(h_ref, mp_ref, W1_ref, b1_ref, g1_ref, be1_ref, W2_ref, b2_ref,
             o_ref):
        hs = h_ref[...] + mp_ref[0, :N, :] + mp_ref[1, :N, :]
        t = jnp.dot(hs, W1_ref[...], preferred_element_type=jnp.float32)
        t = jax.nn.relu(_bn(t + b1_ref[...], g1_ref[...], be1_ref[...]))
        o_ref[...] = jax.nn.relu(
            jnp.dot(t, W2_ref[...], preferred_element_type=jnp.float32)
            + b2_ref[...])
    return pl.pallas_call(
        body, out_shape=jax.ShapeDtypeStruct((N, H), jnp.float32),
    )(h, mp, W1, b1, g1, be1, W2, b2)


def _tc_attnprep(h, eaW1, eaW2, beta, naW, nab, bcg, bcb, bog, bob, ccW, ocW):
    def body(h_ref, eaW1_ref, eaW2_ref, beta_ref, naW_ref, nab_ref,
             bcg_ref, bcb_ref, bog_ref, bob_ref, ccW_ref, ocW_ref,
             u_ref, v_ref, gc_ref, go_ref):
        h = h_ref[...]
        pq = jnp.dot(h, eaW1_ref[...], preferred_element_type=jnp.float32)
        qq = jnp.dot(h, eaW2_ref[...], preferred_element_type=jnp.float32)
        u_ref[...] = pq[:, 0:1] - pq[:, 1:2] + beta_ref[0, 0]
        v_ref[...] = qq[:, 0:1] - qq[:, 1:2]
        nl = jnp.dot(h, naW_ref[...], preferred_element_type=jnp.float32) \
            + nab_ref[...]
        na0 = 1.0 / (1.0 + jnp.exp(nl[:, 1:2] - nl[:, 0:1]))
        xc = na0 * h
        xo = (1.0 - na0) * h
        gc_ref[...] = jnp.dot(_bn(xc, bcg_ref[...], bcb_ref[...]),
                              ccW_ref[...], preferred_element_type=jnp.float32)
        go_ref[...] = jnp.dot(_bn(xo, bog_ref[...], bob_ref[...]),
                              ocW_ref[...], preferred_element_type=jnp.float32)
    return pl.pallas_call(
        body, out_shape=(
            jax.ShapeDtypeStruct((N, 1), jnp.float32),
            jax.ShapeDtypeStruct((N, 1), jnp.float32),
            jax.ShapeDtypeStruct((N, H), jnp.float32),
            jax.ShapeDtypeStruct((N, H), jnp.float32),
        ),
    )(h, eaW1, eaW2, beta, naW, nab, bcg, bcb, bog, bob, ccW, ocW)


def _tc_disprep(degT, gc, go):
    def body(dt_ref, gc_ref, go_ref, gsc_ref, gso_ref, dc_ref, do_ref):
        dt = dt_ref[...]
        lane = lax.broadcasted_iota(jnp.int32, (NPAD, NW * 2), 1)
        evens = jnp.where(lane % 2 == 0, dt, 0.0)
        odds = jnp.where(lane % 2 == 1, dt, 0.0)
        deg_c = jnp.sum(evens, axis=1, keepdims=True)[:N]
        deg_o = jnp.sum(odds, axis=1, keepdims=True)[:N]
        dis_c = lax.rsqrt(deg_c + 1.0)
        dis_o = lax.rsqrt(deg_o + 1.0)
        dc_ref[...] = dis_c
        do_ref[...] = dis_o
        gsc_ref[...] = dis_c * gc_ref[...]
        gso_ref[...] = dis_o * go_ref[...]
    return pl.pallas_call(
        body, out_shape=(
            jax.ShapeDtypeStruct((N, H), jnp.float32),
            jax.ShapeDtypeStruct((N, H), jnp.float32),
            jax.ShapeDtypeStruct((N, 1), jnp.float32),
            jax.ShapeDtypeStruct((N, 1), jnp.float32),
        ),
    )(degT, gc, go)


def _tc_gcnpost(mp_c, mp_o, gsc, gso, dis_c, dis_o, ccb, ocb):
    def body(mpc_ref, mpo_ref, gsc_ref, gso_ref, dc_ref, do_ref,
             ccb_ref, ocb_ref, xc_ref, xo_ref):
        xc_ref[...] = jax.nn.relu(
            dc_ref[...] * (mpc_ref[0, :N, :] + mpc_ref[1, :N, :] + gsc_ref[...])
            + ccb_ref[...])
        xo_ref[...] = jax.nn.relu(
            do_ref[...] * (mpo_ref[0, :N, :] + mpo_ref[1, :N, :] + gso_ref[...])
            + ocb_ref[...])
    return pl.pallas_call(
        body, out_shape=(
            jax.ShapeDtypeStruct((N, H), jnp.float32),
            jax.ShapeDtypeStruct((N, H), jnp.float32),
        ),
    )(mp_c, mp_o, gsc, gso, dis_c, dis_o, ccb, ocb)


def _pad_nodes(h):
    return jnp.pad(h, ((0, NPAD - N), (0, 0)))


def _gin(h, row_p, col_p, p):
    mp = _sc_msgpass(_pad_nodes(h), row_p, col_p)
    return _tc_gin_dense(h, mp, p["W1"], p["b1"], p["g1"], p["be1"],
                         p["W2"], p["b2"])


def kernel(x, edge_index, batch, params):
    p = params
    row, col = edge_index[0], edge_index[1]
    h = _tc_feat(x, p["bn_feat_g"], p["bn_feat_b"],
                 p["conv_feat_W"], p["conv_feat_b"])
    for lp in p["gin"]:
        h = _gin(h, row, col, lp)
    beta = (p["ea_b"][0] - p["ea_b"][1]).reshape(1, 1)
    u, v, gc, go = _tc_attnprep(
        h, p["ea_W"][:H], p["ea_W"][H:], beta, p["na_W"], p["na_b"],
        p["bnc_g"], p["bnc_b"], p["bno_g"], p["bno_b"], p["cc_W"], p["oc_W"])
    ewc, ewo, degtab = _sc_attn(jnp.pad(u[:, 0], (0, NPAD - N)),
                                jnp.pad(v[:, 0], (0, NPAD - N)), row, col)
    degT = degtab.reshape(NW * 2, NPAD).T
    gsc, gso, dis_c, dis_o = _tc_disprep(degT, gc, go)
    mp_c = _sc_msgpass_scaled(_pad_nodes(gsc), row, col, ewc)
    mp_o = _sc_msgpass_scaled(_pad_nodes(gso), row, col, ewo)
    xc, xo = _tc_gcnpost(mp_c, mp_o, gsc, gso, dis_c, dis_o,
                         p["cc_b"], p["oc_b"])
    outc, outo, outco = _pool_and_heads(xc, xo, batch, p)
    return (outc, outo, outco)


# speedup vs baseline: 1.0701x; 1.0701x over previous
import functools

import jax
import jax.numpy as jnp
from jax import lax
from jax.experimental import pallas as pl
from jax.experimental.pallas import tpu as pltpu
from jax.experimental.pallas import tpu_sc as plsc

N = 10000
E = 320000
D = 128
H = 128
C = 10
G = 128
EPS = 1e-5

NC = 2
NS = 16
NW = NC * NS
NPAD = 10240
EW = E // NW
K = 80
STEPS = EW // K


def _scale_rows(rows, sbuf):
    @pl.loop(0, K // 16)
    def _(t):
        s16 = sbuf[pl.ds(t * 16, 16)]
        for l in range(16):
            sc = s16[l]
            for j in range(8):
                rows[t * 16 + l, pl.ds(j * 16, 16)] = (
                    rows[t * 16 + l, pl.ds(j * 16, 16)] * sc)


def _msg_body(scaled, *refs):
    if scaled:
        (h_hbm, row_hbm, col_hbm, ew_hbm, out_hbm, acc_sh,
         ridx_a, cidx_a, sbuf_a, ridx_b, cidx_b, sbuf_b, rows_a, rows_b,
         isem_a, isem_b, gsem_a, gsem_b, ssem_a, ssem_b) = refs
    else:
        (h_hbm, row_hbm, col_hbm, out_hbm, acc_sh,
         ridx_a, cidx_a, ridx_b, cidx_b, rows_a, rows_b,
         isem_a, isem_b, gsem_a, gsem_b, ssem_a, ssem_b) = refs
    cid = lax.axis_index("c")
    sid = lax.axis_index("s")
    wid = sid * NC + cid
    @pl.loop(0, K)
    def _(r):
        for j in range(8):
            rows_a[r, pl.ds(j * 16, 16)] = jnp.zeros((16,), jnp.float32)
    slab = NPAD // NS
    @pl.loop(0, slab // K)
    def _(t):
        pltpu.sync_copy(rows_a, acc_sh.at[pl.ds(sid * slab + t * K, K), :])
    plsc.subcore_barrier()

    def fetch_idx(base, ridx, cidx, sbuf, sem):
        ds = [pltpu.async_copy(row_hbm.at[pl.ds(base, K)], ridx, sem),
              pltpu.async_copy(col_hbm.at[pl.ds(base, K)], cidx, sem)]
        if scaled:
            ds.append(pltpu.async_copy(ew_hbm.at[pl.ds(base, K)], sbuf, sem))
        return ds

    def drain(ds):
        for d in ds:
            d.wait()

    base0 = wid * EW
    @pl.loop(0, STEPS // 2)
    def _(ip):
        base_a = base0 + ip * (2 * K)
        base_b = base_a + K
        ia = fetch_idx(base_a, ridx_a, cidx_a, sbuf_a if scaled else None,
                       isem_a)
        ib = fetch_idx(base_b, ridx_b, cidx_b, sbuf_b if scaled else None,
                       isem_b)
        drain(ia)
        ga = pltpu.async_copy(h_hbm.at[ridx_a], rows_a, gsem_a)
        drain(ib)
        gb = pltpu.async_copy(h_hbm.at[ridx_b], rows_b, gsem_b)
        ga.wait()
        if scaled:
            _scale_rows(rows_a, sbuf_a)
        sa = pltpu.async_copy(rows_a, acc_sh.at[cidx_a], ssem_a, add=True)
        gb.wait()
        if scaled:
            _scale_rows(rows_b, sbuf_b)
        sb = pltpu.async_copy(rows_b, acc_sh.at[cidx_b], ssem_b, add=True)
        sa.wait()
        sb.wait()
    if STEPS % 2:
        base_t = base0 + (STEPS - 1) * K
        drain(fetch_idx(base_t, ridx_a, cidx_a, sbuf_a if scaled else None,
                        isem_a))
        pltpu.async_copy(h_hbm.at[ridx_a], rows_a, gsem_a).wait()
        if scaled:
            _scale_rows(rows_a, sbuf_a)
        pltpu.sync_copy(rows_a, acc_sh.at[cidx_a], add=True)
    plsc.subcore_barrier()
    pltpu.sync_copy(acc_sh.at[pl.ds(sid * slab, slab), :],
                    out_hbm.at[cid, pl.ds(sid * slab, slab), :])


def _attn_body(u_hbm, v_hbm, row_hbm, col_hbm,
               ewc_hbm, ewo_hbm, deg_hbm,
               u_vmem, v_vmem, dc_vmem, do_vmem,
               ridx_a, cidx_a, ridx_b, cidx_b,
               wc_a, wo_a, wc_b, wo_b, isem_a, isem_b, osem):
    cid = lax.axis_index("c")
    sid = lax.axis_index("s")
    wid = sid * NC + cid
    pltpu.sync_copy(u_hbm, u_vmem)
    pltpu.sync_copy(v_hbm, v_vmem)
    @pl.loop(0, NPAD // 16)
    def _(t):
        dc_vmem[pl.ds(t * 16, 16)] = jnp.zeros((16,), jnp.float32)
        do_vmem[pl.ds(t * 16, 16)] = jnp.zeros((16,), jnp.float32)
    base0 = wid * EW

    def compute(ridx, cidx, wc_buf, wo_buf):
        @pl.loop(0, K // 16)
        def _(t):
            r16 = ridx[pl.ds(t * 16, 16)]
            c16 = cidx[pl.ds(t * 16, 16)]
            s = plsc.load_gather(u_vmem, [r16]) + plsc.load_gather(v_vmem, [c16])
            wc = 1.0 / (1.0 + jnp.exp(-s))
            wo = 1.0 - wc
            wc_buf[pl.ds(t * 16, 16)] = wc
            wo_buf[pl.ds(t * 16, 16)] = wo
            plsc.addupdate_scatter(dc_vmem, [r16], wc)
            plsc.addupdate_scatter(do_vmem, [r16], wo)

    @pl.loop(0, STEPS // 2)
    def _(ip):
        base_a = base0 + ip * (2 * K)
        base_b = base_a + K
        ia = [pltpu.async_copy(row_hbm.at[pl.ds(base_a, K)], ridx_a, isem_a),
              pltpu.async_copy(col_hbm.at[pl.ds(base_a, K)], cidx_a, isem_a)]
        ib = [pltpu.async_copy(row_hbm.at[pl.ds(base_b, K)], ridx_b, isem_b),
              pltpu.async_copy(col_hbm.at[pl.ds(base_b, K)], cidx_b, isem_b)]
        for d in ia:
            d.wait()
        compute(ridx_a, cidx_a, wc_a, wo_a)
        oa = [pltpu.async_copy(wc_a, ewc_hbm.at[pl.ds(base_a, K)], osem),
              pltpu.async_copy(wo_a, ewo_hbm.at[pl.ds(base_a, K)], osem)]
        for d in ib:
            d.wait()
        compute(ridx_b, cidx_b, wc_b, wo_b)
        ob = [pltpu.async_copy(wc_b, ewc_hbm.at[pl.ds(base_b, K)], osem),
              pltpu.async_copy(wo_b, ewo_hbm.at[pl.ds(base_b, K)], osem)]
        for d in oa + ob:
            d.wait()
    if STEPS % 2:
        base_t = base0 + (STEPS - 1) * K
        pltpu.sync_copy(row_hbm.at[pl.ds(base_t, K)], ridx_a)
        pltpu.sync_copy(col_hbm.at[pl.ds(base_t, K)], cidx_a)
        compute(ridx_a, cidx_a, wc_a, wo_a)
        pltpu.sync_copy(wc_a, ewc_hbm.at[pl.ds(base_t, K)])
        pltpu.sync_copy(wo_a, ewo_hbm.at[pl.ds(base_t, K)])
    pltpu.sync_copy(dc_vmem, deg_hbm.at[wid, 0])
    pltpu.sync_copy(do_vmem, deg_hbm.at[wid, 1])


@jax.jit
def _sc_attn(u_pad, v_pad, row, col):
    mesh = plsc.VectorSubcoreMesh(core_axis_name="c", subcore_axis_name="s")
    kern = pl.kernel(
        _attn_body,
        compiler_params=pltpu.CompilerParams(needs_layout_passes=False),
        out_type=(
            jax.ShapeDtypeStruct((E,), jnp.float32),
            jax.ShapeDtypeStruct((E,), jnp.float32),
            jax.ShapeDtypeStruct((NW, 2, NPAD), jnp.float32),
        ),
        mesh=mesh,
        scratch_types=[
            pltpu.VMEM((NPAD,), jnp.float32),
            pltpu.VMEM((NPAD,), jnp.float32),
            pltpu.VMEM((NPAD,), jnp.float32),
            pltpu.VMEM((NPAD,), jnp.float32),
            pltpu.VMEM((K,), jnp.int32),
            pltpu.VMEM((K,), jnp.int32),
            pltpu.VMEM((K,), jnp.int32),
            pltpu.VMEM((K,), jnp.int32),
            pltpu.VMEM((K,), jnp.float32),
            pltpu.VMEM((K,), jnp.float32),
            pltpu.VMEM((K,), jnp.float32),
            pltpu.VMEM((K,), jnp.float32),
        ] + [pltpu.SemaphoreType.DMA] * 3,
    )
    return kern(u_pad, v_pad, row, col)


@jax.jit
def _sc_msgpass(h_pad, row, col):
    mesh = plsc.VectorSubcoreMesh(core_axis_name="c", subcore_axis_name="s")
    kern = pl.kernel(
        functools.partial(_msg_body, False),
        out_type=jax.ShapeDtypeStruct((NC, NPAD, 128), jnp.float32),
        mesh=mesh,
        scratch_types=[
            pltpu.VMEM_SHARED((NPAD, 128), jnp.float32),
            pltpu.VMEM((K,), jnp.int32),
            pltpu.VMEM((K,), jnp.int32),
            pltpu.VMEM((K,), jnp.int32),
            pltpu.VMEM((K,), jnp.int32),
            pltpu.VMEM((K, 128), jnp.float32),
            pltpu.VMEM((K, 128), jnp.float32),
        ] + [pltpu.SemaphoreType.DMA] * 6,
    )
    return kern(h_pad, row, col)


@jax.jit
def _sc_msgpass_scaled(h_pad, row, col, ew):
    mesh = plsc.VectorSubcoreMesh(core_axis_name="c", subcore_axis_name="s")
    kern = pl.kernel(
        functools.partial(_msg_body, True),
        out_type=jax.ShapeDtypeStruct((NC, NPAD, 128), jnp.float32),
        mesh=mesh,
        scratch_types=[
            pltpu.VMEM_SHARED((NPAD, 128), jnp.float32),
            pltpu.VMEM((K,), jnp.int32),
            pltpu.VMEM((K,), jnp.int32),
            pltpu.VMEM((K,), jnp.float32),
            pltpu.VMEM((K,), jnp.int32),
            pltpu.VMEM((K,), jnp.int32),
            pltpu.VMEM((K,), jnp.float32),
            pltpu.VMEM((K, 128), jnp.float32),
            pltpu.VMEM((K, 128), jnp.float32),
        ] + [pltpu.SemaphoreType.DMA] * 6,
    )
    return kern(h_pad, row, col, ew)


def _bn(x, g, b):
    m = jnp.mean(x, axis=0)
    v = jnp.mean(x * x, axis=0) - m * m
    return (x - m) * lax.rsqrt(v + EPS) * g + b


def _log_softmax(z):
    zm = z - jnp.max(z, axis=-1, keepdims=True)
    return zm - jnp.log(jnp.sum(jnp.exp(zm), axis=-1, keepdims=True))


def _head(z, p, pre):
    z = _bn(z, p[pre + "1bn_g"], p[pre + "1bn_b"])
    z = jax.nn.relu(z @ p[pre + "1_W"] + p[pre + "1_b"])
    z = _bn(z, p[pre + "2bn_g"], p[pre + "2bn_b"])
    z = z @ p[pre + "2_W"] + p[pre + "2_b"]
    return _log_softmax(z)


def _pool_heads_body(xc_ref, xo_ref, batch_ref, *rest):
    (hp_refs, outc_ref, outo_ref, outco_ref) = (rest[:-3], rest[-3], rest[-2], rest[-1])
    names = _HEAD_PARAM_NAMES
    p = {k: r[...] for k, r in zip(names, hp_refs)}
    onehot = (batch_ref[0:1, :] == lax.broadcasted_iota(jnp.int32, (G, N), 0))
    onehot = onehot.astype(jnp.float32)
    pc = jnp.dot(onehot, xc_ref[...], preferred_element_type=jnp.float32)
    po = jnp.dot(onehot, xo_ref[...], preferred_element_type=jnp.float32)
    outc_ref[...] = _head(pc, p, "c")
    outo_ref[...] = _head(po, p, "o")
    outco_ref[...] = _head(pc + po, p, "co")


_HEAD_PARAM_NAMES = tuple(
    pre + suf
    for pre in ("c", "o", "co")
    for suf in ("1bn_g", "1bn_b", "1_W", "1_b", "2bn_g", "2bn_b", "2_W", "2_b")
)


def _pool_and_heads(xc, xo, batch, params):
    hp = [params[k] for k in _HEAD_PARAM_NAMES]
    out_shape = [jax.ShapeDtypeStruct((G, C), jnp.float32)] * 3
    outs = pl.pallas_call(
        _pool_heads_body,
        out_shape=out_shape,
    )(xc, xo, batch.reshape(1, N), *hp)
    return outs


def _tc_feat(x, g, b, W, wb):
    def body(x_ref, g_ref, b_ref, W_ref, wb_ref, o_ref):
        h = _bn(x_ref[...], g_ref[...], b_ref[...])
        o_ref[...] = jax.nn.relu(
            jnp.dot(h, W_ref[...], preferred_element_type=jnp.float32)
            + wb_ref[...])
    return pl.pallas_call(
        body, out_shape=jax.ShapeDtypeStruct((N, H), jnp.float32),
    )(x, g, b, W, wb)


def _tc_gin_dense(h, mp, W1, b1, g1, be1, W2, b2):
    def body(h_ref, mp_ref, W1_ref, b1_ref, g1_ref, be1_ref, W2_ref, b2_ref,
             o_ref):
        hs = h_ref[...] + mp_ref[0, :N, :] + mp_ref[1, :N, :]
        t = jnp.dot(hs, W1_ref[...], preferred_element_type=jnp.float32)
        t = jax.nn.relu(_bn(t + b1_ref[...], g1_ref[...], be1_ref[...]))
        o_ref[...] = jax.nn.relu(
            jnp.dot(t, W2_ref[...], preferred_element_type=jnp.float32)
            + b2_ref[...])
    return pl.pallas_call(
        body, out_shape=jax.ShapeDtypeStruct((N, H), jnp.float32),
    )(h, mp, W1, b1, g1, be1, W2, b2)


def _tc_attnprep(h, eaW1, eaW2, beta, naW, nab, bcg, bcb, bog, bob, ccW, ocW):
    def body(h_ref, eaW1_ref, eaW2_ref, beta_ref, naW_ref, nab_ref,
             bcg_ref, bcb_ref, bog_ref, bob_ref, ccW_ref, ocW_ref,
             u_ref, v_ref, gc_ref, go_ref):
        h = h_ref[...]
        pq = jnp.dot(h, eaW1_ref[...], preferred_element_type=jnp.float32)
        qq = jnp.dot(h, eaW2_ref[...], preferred_element_type=jnp.float32)
        u_ref[...] = pq[:, 0:1] - pq[:, 1:2] + beta_ref[0, 0]
        v_ref[...] = qq[:, 0:1] - qq[:, 1:2]
        nl = jnp.dot(h, naW_ref[...], preferred_element_type=jnp.float32) \
            + nab_ref[...]
        na0 = 1.0 / (1.0 + jnp.exp(nl[:, 1:2] - nl[:, 0:1]))
        xc = na0 * h
        xo = (1.0 - na0) * h
        gc_ref[...] = jnp.dot(_bn(xc, bcg_ref[...], bcb_ref[...]),
                              ccW_ref[...], preferred_element_type=jnp.float32)
        go_ref[...] = jnp.dot(_bn(xo, bog_ref[...], bob_ref[...]),
                              ocW_ref[...], preferred_element_type=jnp.float32)
    return pl.pallas_call(
        body, out_shape=(
            jax.ShapeDtypeStruct((N, 1), jnp.float32),
            jax.ShapeDtypeStruct((N, 1), jnp.float32),
            jax.ShapeDtypeStruct((N, H), jnp.float32),
            jax.ShapeDtypeStruct((N, H), jnp.float32),
        ),
    )(h, eaW1, eaW2, beta, naW, nab, bcg, bcb, bog, bob, ccW, ocW)


def _tc_disprep(degT, gc, go):
    def body(dt_ref, gc_ref, go_ref, gsc_ref, gso_ref, dc_ref, do_ref):
        dt = dt_ref[...]
        lane = lax.broadcasted_iota(jnp.int32, (NPAD, NW * 2), 1)
        evens = jnp.where(lane % 2 == 0, dt, 0.0)
        odds = jnp.where(lane % 2 == 1, dt, 0.0)
        deg_c = jnp.sum(evens, axis=1, keepdims=True)[:N]
        deg_o = jnp.sum(odds, axis=1, keepdims=True)[:N]
        dis_c = lax.rsqrt(deg_c + 1.0)
        dis_o = lax.rsqrt(deg_o + 1.0)
        dc_ref[...] = dis_c
        do_ref[...] = dis_o
        gsc_ref[...] = dis_c * gc_ref[...]
        gso_ref[...] = dis_o * go_ref[...]
    return pl.pallas_call(
        body, out_shape=(
            jax.ShapeDtypeStruct((N, H), jnp.float32),
            jax.ShapeDtypeStruct((N, H), jnp.float32),
            jax.ShapeDtypeStruct((N, 1), jnp.float32),
            jax.ShapeDtypeStruct((N, 1), jnp.float32),
        ),
    )(degT, gc, go)


def _tc_gcnpost(mp_c, mp_o, gsc, gso, dis_c, dis_o, ccb, ocb):
    def body(mpc_ref, mpo_ref, gsc_ref, gso_ref, dc_ref, do_ref,
             ccb_ref, ocb_ref, xc_ref, xo_ref):
        xc_ref[...] = jax.nn.relu(
            dc_ref[...] * (mpc_ref[0, :N, :] + mpc_ref[1, :N, :] + gsc_ref[...])
            + ccb_ref[...])
        xo_ref[...] = jax.nn.relu(
            do_ref[...] * (mpo_ref[0, :N, :] + mpo_ref[1, :N, :] + gso_ref[...])
            + ocb_ref[...])
    return pl.pallas_call(
        body, out_shape=(
            jax.ShapeDtypeStruct((N, H), jnp.float32),
            jax.ShapeDtypeStruct((N, H), jnp.float32),
        ),
    )(mp_c, mp_o, gsc, gso, dis_c, dis_o, ccb, ocb)


def _pad_nodes(h):
    return jnp.pad(h, ((0, NPAD - N), (0, 0)))


def _gin(h, row_p, col_p, p):
    mp = _sc_msgpass(_pad_nodes(h), row_p, col_p)
    return _tc_gin_dense(h, mp, p["W1"], p["b1"], p["g1"], p["be1"],
                         p["W2"], p["b2"])


def kernel(x, edge_index, batch, params):
    p = params
    row, col = edge_index[0], edge_index[1]
    h = _tc_feat(x, p["bn_feat_g"], p["bn_feat_b"],
                 p["conv_feat_W"], p["conv_feat_b"])
    for lp in p["gin"]:
        h = _gin(h, row, col, lp)
    beta = (p["ea_b"][0] - p["ea_b"][1]).reshape(1, 1)
    u, v, gc, go = _tc_attnprep(
        h, p["ea_W"][:H], p["ea_W"][H:], beta, p["na_W"], p["na_b"],
        p["bnc_g"], p["bnc_b"], p["bno_g"], p["bno_b"], p["cc_W"], p["oc_W"])
    ewc, ewo, degtab = _sc_attn(jnp.pad(u[:, 0], (0, NPAD - N)),
                                jnp.pad(v[:, 0], (0, NPAD - N)), row, col)
    degT = degtab.reshape(NW * 2, NPAD).T
    gsc, gso, dis_c, dis_o = _tc_disprep(degT, gc, go)
    mp_c = _sc_msgpass_scaled(_pad_nodes(gsc), row, col, ewc)
    mp_o = _sc_msgpass_scaled(_pad_nodes(gso), row, col, ewo)
    xc, xo = _tc_gcnpost(mp_c, mp_o, gsc, gso, dis_c, dis_o,
                         p["cc_b"], p["oc_b"])
    outc, outo, outco = _pool_and_heads(xc, xo, batch, p)
    return (outc, outo, outco)
